# baseline (speedup 1.0000x reference)
import functools

import jax
import jax.numpy as jnp
from jax import lax
from jax.experimental import pallas as pl
from jax.experimental.pallas import tpu as pltpu
from jax.experimental.pallas import tpu_sc as plsc

_info = plsc.get_sparse_core_info()
_NC, _NS, _L = _info.num_cores, _info.num_subcores, _info.num_lanes
_NW = _NC * _NS

_IDX = 64


def _build(b, s, d):
    chunk = b * s // _NW
    nh = chunk // _IDX
    wpb = s // chunk
    assert chunk * _NW == b * s and nh * _IDX == chunk and wpb * chunk == s
    mesh = plsc.VectorSubcoreMesh(core_axis_name="c", subcore_axis_name="s")

    @functools.partial(
        pl.kernel,
        mesh=mesh,
        out_type=jax.ShapeDtypeStruct((b, s, d), jnp.float32),
        scratch_types=[
            pltpu.VMEM((nh, _IDX), jnp.int32),
            pltpu.VMEM((nh, _IDX, d), jnp.float32),
            pltpu.SemaphoreType.DMA((nh,)),
            pltpu.SemaphoreType.DMA((nh,)),
            pltpu.SemaphoreType.DMA((nh,)),
            pltpu.SemaphoreType.DMA((nh,)),
        ],
    )
    def k(x_hbm, table_hbm, pos_hbm, out_hbm, idx_v, rows_v,
          sem_p, sem_i, sem_g, sem_w):
        wid = lax.axis_index("s") * _NC + lax.axis_index("c")
        bi = wid // wpb
        p0 = (wid % wpb) * chunk
        pos_cps = [
            pltpu.async_copy(
                pos_hbm.at[0, pl.ds(p0 + h * _IDX, _IDX)], rows_v.at[h],
                sem_p.at[h],
            )
            for h in range(nh)
        ]
        i_cps = [
            pltpu.async_copy(
                x_hbm.at[bi, pl.ds(p0 + h * _IDX, _IDX)], idx_v.at[h],
                sem_i.at[h],
            )
            for h in range(nh)
        ]
        g_cps = []
        for h in range(nh):
            pos_cps[h].wait()
            i_cps[h].wait()
            g_cps.append(
                pltpu.async_copy(
                    table_hbm.at[idx_v.at[h]], rows_v.at[h], sem_g.at[h],
                    add=True,
                )
            )
        w_cps = []
        for h in range(nh):
            g_cps[h].wait()
            w_cps.append(
                pltpu.async_copy(
                    rows_v.at[h], out_hbm.at[bi, pl.ds(p0 + h * _IDX, _IDX)],
                    sem_w.at[h],
                )
            )
        for cp in w_cps:
            cp.wait()

    return k


def kernel(X, token_table, pos_embed):
    b, s = X.shape
    vocab, d = token_table.shape
    return _build(b, s, d)(X.astype(jnp.int32), token_table, pos_embed)


# speedup vs baseline: 1.0124x; 1.0124x over previous
import functools

import jax
import jax.numpy as jnp
from jax import lax
from jax.experimental import pallas as pl
from jax.experimental.pallas import tpu as pltpu
from jax.experimental.pallas import tpu_sc as plsc

_info = plsc.get_sparse_core_info()
_NC, _NS, _L = _info.num_cores, _info.num_subcores, _info.num_lanes
_NW = _NC * _NS

_IDX = 256


def _build(b, s, d):
    chunk = b * s // _NW
    nh = chunk // _IDX
    wpb = s // chunk
    assert chunk * _NW == b * s and nh * _IDX == chunk and wpb * chunk == s
    mesh = plsc.VectorSubcoreMesh(core_axis_name="c", subcore_axis_name="s")

    @functools.partial(
        pl.kernel,
        mesh=mesh,
        out_type=jax.ShapeDtypeStruct((b, s, d), jnp.float32),
        scratch_types=[
            pltpu.VMEM((nh, _IDX), jnp.int32),
            pltpu.VMEM((nh, _IDX, d), jnp.float32),
            pltpu.SemaphoreType.DMA((nh,)),
            pltpu.SemaphoreType.DMA((nh,)),
            pltpu.SemaphoreType.DMA((nh,)),
            pltpu.SemaphoreType.DMA((nh,)),
        ],
    )
    def k(x_hbm, table_hbm, pos_hbm, out_hbm, idx_v, rows_v,
          sem_p, sem_i, sem_g, sem_w):
        wid = lax.axis_index("s") * _NC + lax.axis_index("c")
        bi = wid // wpb
        p0 = (wid % wpb) * chunk
        pos_cps = [
            pltpu.async_copy(
                pos_hbm.at[0, pl.ds(p0 + h * _IDX, _IDX)], rows_v.at[h],
                sem_p.at[h],
            )
            for h in range(nh)
        ]
        i_cps = [
            pltpu.async_copy(
                x_hbm.at[bi, pl.ds(p0 + h * _IDX, _IDX)], idx_v.at[h],
                sem_i.at[h],
            )
            for h in range(nh)
        ]
        g_cps = []
        for h in range(nh):
            pos_cps[h].wait()
            i_cps[h].wait()
            g_cps.append(
                pltpu.async_copy(
                    table_hbm.at[idx_v.at[h]], rows_v.at[h], sem_g.at[h],
                    add=True,
                )
            )
        w_cps = []
        for h in range(nh):
            g_cps[h].wait()
            w_cps.append(
                pltpu.async_copy(
                    rows_v.at[h], out_hbm.at[bi, pl.ds(p0 + h * _IDX, _IDX)],
                    sem_w.at[h],
                )
            )
        for cp in w_cps:
            cp.wait()

    return k


def kernel(X, token_table, pos_embed):
    b, s = X.shape
    vocab, d = token_table.shape
    return _build(b, s, d)(X.astype(jnp.int32), token_table, pos_embed)


# speedup vs baseline: 1.0214x; 1.0089x over previous
import functools

import jax
import jax.numpy as jnp
from jax import lax
from jax.experimental import pallas as pl
from jax.experimental.pallas import tpu as pltpu
from jax.experimental.pallas import tpu_sc as plsc

_info = plsc.get_sparse_core_info()
_NC, _NS, _L = _info.num_cores, _info.num_subcores, _info.num_lanes
_NW = _NC * _NS

_IDX = 128


def _build(b, s, d):
    chunk = b * s // _NW
    nh = chunk // _IDX
    wpb = s // chunk
    assert chunk * _NW == b * s and nh * _IDX == chunk and wpb * chunk == s
    mesh = plsc.VectorSubcoreMesh(core_axis_name="c", subcore_axis_name="s")

    @functools.partial(
        pl.kernel,
        mesh=mesh,
        out_type=jax.ShapeDtypeStruct((b, s, d), jnp.float32),
        scratch_types=[
            pltpu.VMEM((nh, _IDX), jnp.int32),
            pltpu.VMEM((nh, _IDX, d), jnp.float32),
            pltpu.SemaphoreType.DMA((nh,)),
            pltpu.SemaphoreType.DMA((nh,)),
            pltpu.SemaphoreType.DMA((nh,)),
            pltpu.SemaphoreType.DMA((nh,)),
        ],
    )
    def k(x_hbm, table_hbm, pos_hbm, out_hbm, idx_v, rows_v,
          sem_p, sem_i, sem_g, sem_w):
        wid = lax.axis_index("s") * _NC + lax.axis_index("c")
        bi = wid // wpb
        p0 = (wid % wpb) * chunk
        pos_cps = [
            pltpu.async_copy(
                pos_hbm.at[0, pl.ds(p0 + h * _IDX, _IDX)], rows_v.at[h],
                sem_p.at[h],
            )
            for h in range(nh)
        ]
        i_cps = [
            pltpu.async_copy(
                x_hbm.at[bi, pl.ds(p0 + h * _IDX, _IDX)], idx_v.at[h],
                sem_i.at[h],
            )
            for h in range(nh)
        ]
        g_cps = []
        for h in range(nh):
            pos_cps[h].wait()
            i_cps[h].wait()
            g_cps.append(
                pltpu.async_copy(
                    table_hbm.at[idx_v.at[h]], rows_v.at[h], sem_g.at[h],
                    add=True,
                )
            )
        w_cps = []
        for h in range(nh):
            g_cps[h].wait()
            w_cps.append(
                pltpu.async_copy(
                    rows_v.at[h], out_hbm.at[bi, pl.ds(p0 + h * _IDX, _IDX)],
                    sem_w.at[h],
                )
            )
        for cp in w_cps:
            cp.wait()

    return k


def kernel(X, token_table, pos_embed):
    b, s = X.shape
    vocab, d = token_table.shape
    return _build(b, s, d)(X.astype(jnp.int32), token_table, pos_embed)
